# speedup vs baseline: 29.3887x; 29.3887x over previous
import functools
import math

import jax
import jax.numpy as jnp
from jax import lax
from jax.experimental import pallas as pl
from jax.experimental.pallas import tpu as pltpu
from jax.experimental.pallas import tpu_sc as plsc

_NC = 2
_NS = 16
_NW = _NC * _NS
_LANES = 16


@functools.lru_cache(maxsize=None)
def _make_sc_gather_mean(M, K, D):
    npw = M // _NW
    G = 2
    nchunks = npw // G
    nvec = D // _LANES
    inv_k = 1.0 / K
    mesh = plsc.VectorSubcoreMesh(core_axis_name="c", subcore_axis_name="s")

    @functools.partial(
        pl.kernel,
        out_type=jax.ShapeDtypeStruct((M, D), jnp.float32),
        mesh=mesh,
        scratch_types=[
            pltpu.VMEM((npw * K,), jnp.int32),
            pltpu.VMEM((2, G * K, D), jnp.float32),
            pltpu.VMEM((npw, D), jnp.float32),
            pltpu.SemaphoreType.DMA,
            pltpu.SemaphoreType.DMA,
        ],
    )
    def sc_gather_mean(x_hbm, adj_hbm, out_hbm, idx_v, rows_v, out_v, sem0, sem1):
        cid = lax.axis_index("c")
        sid = lax.axis_index("s")
        wid = sid * _NC + cid
        base = wid * npw
        sems = (sem0, sem1)

        pltpu.sync_copy(adj_hbm.at[pl.ds(base * K, npw * K)], idx_v)

        for b2 in range(2):
            pltpu.async_copy(
                x_hbm.at[idx_v.at[pl.ds(b2 * G * K, G * K)]],
                rows_v.at[b2], sems[b2])

        def pair_body(g, carry):
            for b2 in range(2):
                c = 2 * g + b2
                pltpu.make_async_copy(
                    x_hbm.at[idx_v.at[pl.ds(c * G * K, G * K)]],
                    rows_v.at[b2], sems[b2]).wait()
                for n2 in range(G):
                    def jbody(j, accs, _b2=b2, _n2=n2):
                        return tuple(
                            accs[d] + rows_v[_b2, _n2 * K + j, pl.ds(d * _LANES, _LANES)]
                            for d in range(nvec))
                    init = tuple(rows_v[b2, n2 * K, pl.ds(d * _LANES, _LANES)]
                                 for d in range(nvec))
                    accs = lax.fori_loop(1, K, jbody, init)
                    node = c * G + n2
                    for d in range(nvec):
                        out_v[node, pl.ds(d * _LANES, _LANES)] = accs[d] * inv_k

                @pl.when(c + 2 < nchunks)
                def _(_b2=b2, _c=c):
                    pltpu.async_copy(
                        x_hbm.at[idx_v.at[pl.ds((_c + 2) * G * K, G * K)]],
                        rows_v.at[_b2], sems[_b2])
            return carry

        lax.fori_loop(0, nchunks // 2, pair_body, 0)
        pltpu.sync_copy(out_v, out_hbm.at[pl.ds(base, npw)])

    return sc_gather_mean


@functools.lru_cache(maxsize=None)
def _make_tc_tail(M, D, blk):
    inv_sqrt2 = 1.0 / math.sqrt(2.0)

    def body(x_ref, nm_ref, w1_ref, w2_ref, b_ref, g_ref, be_ref, o_ref):
        xb = x_ref[...]
        h = jnp.dot(xb, w1_ref[...], preferred_element_type=jnp.float32)
        h = h + jnp.dot(nm_ref[...], w2_ref[...], preferred_element_type=jnp.float32)
        h = h + b_ref[...]
        h = 0.5 * h * (1.0 + lax.erf(h * inv_sqrt2))
        hh = h + xb
        mu = jnp.mean(hh, axis=1, keepdims=True)
        dv = hh - mu
        var = jnp.mean(dv * dv, axis=1, keepdims=True)
        o_ref[...] = dv * lax.rsqrt(var + 1e-5) * g_ref[...] + be_ref[...]

    return pl.pallas_call(
        body,
        grid=(M // blk,),
        in_specs=[
            pl.BlockSpec((blk, D), lambda i: (i, 0)),
            pl.BlockSpec((blk, D), lambda i: (i, 0)),
            pl.BlockSpec((D, D), lambda i: (0, 0)),
            pl.BlockSpec((D, D), lambda i: (0, 0)),
            pl.BlockSpec((1, D), lambda i: (0, 0)),
            pl.BlockSpec((1, D), lambda i: (0, 0)),
            pl.BlockSpec((1, D), lambda i: (0, 0)),
        ],
        out_specs=pl.BlockSpec((blk, D), lambda i: (i, 0)),
        out_shape=jax.ShapeDtypeStruct((M, D), jnp.float32),
    )


def kernel(x, adj, W, b, gamma, beta):
    B, N, D = x.shape
    K = adj.shape[-1]
    M = B * N
    xf = x.reshape(M, D)
    offs = (jnp.arange(B, dtype=jnp.int32) * N)[:, None, None]
    adjf = (adj.astype(jnp.int32) + offs).reshape(M * K)
    nm = _make_sc_gather_mean(M, K, D)(xf, adjf)
    y = _make_tc_tail(M, D, 1024)(
        xf, nm, W[:D], W[D:], b[None], gamma[None], beta[None])
    return y.reshape(B, N, D)
